# baseline (speedup 1.0000x reference)
import functools

import jax
import jax.numpy as jnp
from jax import lax
from jax.experimental import pallas as pl
from jax.experimental.pallas import tpu as pltpu
from jax.experimental.pallas import tpu_sc as plsc

BATCH = 16384
EMB_K = 64
NUM_CORES = 2
NUM_SUBCORES = 16
NW = NUM_CORES * NUM_SUBCORES
B_PER_W = BATCH // NW


def _sc_gather_body(u_idx_hbm, v_idx_hbm, w_hbm, h_hbm, u_out, v_out,
                    uidx_v, vidx_v, urows_v, vrows_v, usem, vsem):
    wid = lax.axis_index("s") * NUM_CORES + lax.axis_index("c")
    base = wid * B_PER_W
    pltpu.sync_copy(u_idx_hbm.at[pl.ds(base, B_PER_W)], uidx_v)
    pltpu.sync_copy(v_idx_hbm.at[pl.ds(base, B_PER_W)], vidx_v)
    ucp = pltpu.async_copy(w_hbm.at[uidx_v], urows_v, usem)
    vcp = pltpu.async_copy(h_hbm.at[vidx_v], vrows_v, vsem)
    ucp.wait()
    pltpu.sync_copy(urows_v, u_out.at[pl.ds(base, B_PER_W)])
    vcp.wait()
    pltpu.sync_copy(vrows_v, v_out.at[pl.ds(base, B_PER_W)])


@functools.cache
def _sc_gather():
    return pl.kernel(
        _sc_gather_body,
        mesh=plsc.VectorSubcoreMesh(
            core_axis_name="c", subcore_axis_name="s",
            num_cores=NUM_CORES, num_subcores=NUM_SUBCORES),
        out_type=[
            jax.ShapeDtypeStruct((BATCH, EMB_K), jnp.float32),
            jax.ShapeDtypeStruct((BATCH, EMB_K), jnp.float32),
        ],
        scratch_types=[
            pltpu.VMEM((B_PER_W,), jnp.int32),
            pltpu.VMEM((B_PER_W,), jnp.int32),
            pltpu.VMEM((B_PER_W, EMB_K), jnp.float32),
            pltpu.VMEM((B_PER_W, EMB_K), jnp.float32),
            pltpu.SemaphoreType.DMA,
            pltpu.SemaphoreType.DMA,
        ],
        compiler_params=pltpu.CompilerParams(use_tc_tiling_on_sc=False),
    )


BLK = 1024


def _mlp_body(u_ref, v_ref, w1u_ref, w1v_ref, b1_ref, w2_ref, out_ref):
    h = (jnp.dot(u_ref[...], w1u_ref[...], preferred_element_type=jnp.float32)
         + jnp.dot(v_ref[...], w1v_ref[...], preferred_element_type=jnp.float32)
         + b1_ref[...])
    h = jnp.maximum(h, 0.0)
    logit = jnp.sum(h * w2_ref[...], axis=1, keepdims=True)
    out_ref[...] = jax.nn.sigmoid(logit)


def _mlp(u, v, w1u, w1v, b1, w2):
    grid = (BATCH // BLK,)
    return pl.pallas_call(
        _mlp_body,
        grid=grid,
        in_specs=[
            pl.BlockSpec((BLK, EMB_K), lambda i: (i, 0)),
            pl.BlockSpec((BLK, EMB_K), lambda i: (i, 0)),
            pl.BlockSpec((EMB_K, EMB_K), lambda i: (0, 0)),
            pl.BlockSpec((EMB_K, EMB_K), lambda i: (0, 0)),
            pl.BlockSpec((1, EMB_K), lambda i: (0, 0)),
            pl.BlockSpec((1, EMB_K), lambda i: (0, 0)),
        ],
        out_specs=pl.BlockSpec((BLK, 1), lambda i: (i, 0)),
        out_shape=jax.ShapeDtypeStruct((BATCH, 1), jnp.float32),
    )(u, v, w1u, w1v, b1, w2)


def kernel(x, W_table, H_table, W1, b1, W2):
    u_idx = x[:, 0]
    v_idx = x[:, 1]
    u_emb, v_emb = _sc_gather()(u_idx, v_idx, W_table, H_table)
    w1u = W1[:, :EMB_K].T
    w1v = W1[:, EMB_K:].T
    out2d = _mlp(u_emb, v_emb, w1u, w1v, b1.reshape(1, EMB_K), W2)
    return (out2d[:, 0], u_emb, v_emb)
